# baseline (speedup 1.0000x reference)
import jax
import jax.numpy as jnp
import numpy as np
from jax.experimental import pallas as pl
from jax.experimental.pallas import tpu as pltpu

_B = 64
_V = 100000
_K = 50
_RB = 8

_I32_MIN = np.int32(-2147483648)
_I32_MAX = np.int32(2147483647)
_NEG_INF = np.float32(-np.inf)


def _sortable(bits):
    return jnp.where(bits < 0, bits ^ jnp.int32(0x7FFFFFFF), bits)


def _unsortable(s):
    return jnp.where(s < 0, s ^ jnp.int32(0x7FFFFFFF), s)


def _mid(lo, hi):
    return (lo >> 1) + (hi >> 1) + (lo & hi & 1)


def _sampler_block(logits_ref, temp_ref, topk_ref, topp_ref, minp_ref,
                   need_ref, ga_ref, m_ref, rank_ref, out_ref, s_ref, e_ref):
    s_ref[...] = logits_ref[...] / temp_ref[...]
    pos = jax.lax.broadcasted_iota(jnp.int32, (1, _V), 1)

    topk = topk_ref[...]
    topp = topp_ref[...]
    minp = minp_ref[...]
    need = need_ref[...] != 0
    ga = ga_ref[...]
    m_tail = m_ref[...]
    rank = rank_ref[...]

    lane_k = jax.lax.broadcasted_iota(jnp.int32, (1, _K), 1)

    def val_body(_, carry):
        lo, hi, cnt_hi = carry
        mid = _mid(lo, hi)
        mid_f = jax.lax.bitcast_convert_type(_unsortable(mid), jnp.float32)
        cnt = jnp.sum((s_ref[...] > mid_f).astype(jnp.int32),
                      axis=1, keepdims=True)
        le = cnt <= rank
        return (jnp.where(le, lo, mid + 1), jnp.where(le, mid, hi),
                jnp.where(le, cnt, cnt_hi))

    lo0 = jnp.full((_RB, 1), np.int32(-2139095041))
    hi0 = jnp.full((_RB, 1), np.int32(2139095040))
    cnt0 = jnp.zeros((_RB, 1), jnp.int32)
    xstar, _, cnt_gt = jax.lax.fori_loop(0, 32, val_body, (lo0, hi0, cnt0))
    xstar_f = jax.lax.bitcast_convert_type(_unsortable(xstar), jnp.float32)
    k2 = rank - cnt_gt

    e_ref[...] = jnp.where(s_ref[...] == xstar_f, pos, jnp.int32(_V))

    def idx_body(_, carry):
        lo, hi = carry
        mid = _mid(lo, hi)
        cnt = jnp.sum((e_ref[...] <= mid).astype(jnp.int32),
                      axis=1, keepdims=True)
        ge = cnt >= k2 + 1
        return jnp.where(ge, lo, mid + 1), jnp.where(ge, mid, hi)

    lo0i = jnp.zeros((_RB, 1), jnp.int32)
    hi0i = jnp.full((_RB, 1), jnp.int32(131071))
    pstar, _ = jax.lax.fori_loop(0, 17, idx_body, (lo0i, hi0i))
    tstar = jnp.int32(_V - 1) - pstar

    def extract_body(k, carry):
        prev_p, csum, v0, best_s, best_i = carry
        sv = jnp.where(pos == prev_p, _NEG_INF, s_ref[...])
        s_ref[...] = sv
        m = jnp.max(sv, axis=1, keepdims=True)
        pm = jnp.argmax(sv, axis=1, keepdims=True).astype(jnp.int32)
        im = jnp.int32(_V - 1) - pm
        csum = csum + m
        v0 = jnp.where(k == 0, m, v0)
        thr = jnp.where(need, v0 * minp, _NEG_INF)
        keep = (k < topk) & (csum - m <= topp) & (m >= thr)
        mval = jnp.where(keep, m, jnp.float32(0.0))
        gk = jnp.sum(jnp.where(lane_k == k, ga, jnp.float32(0.0)),
                     axis=1, keepdims=True)
        score = mval + gk
        upd = score > best_s
        best_s = jnp.where(upd, score, best_s)
        best_i = jnp.where(upd, im, best_i)
        return pm, csum, v0, best_s, best_i

    zero_col = jnp.zeros((_RB, 1), jnp.float32)
    init = (jnp.full((_RB, 1), jnp.int32(-1)),
            zero_col, zero_col,
            jnp.full((_RB, 1), _NEG_INF),
            jnp.zeros((_RB, 1), jnp.int32))
    _, _, _, best_s, best_i = jax.lax.fori_loop(0, _K, extract_body, init)

    out_ref[...] = jnp.where(best_s >= m_tail, best_i, tstar)


def kernel(logits, temperatures, top_ks, top_ps, min_ps, need_min_p_sampling):
    logits = jnp.reshape(logits, (-1, logits.shape[-1]))

    g = jax.random.gumbel(jax.random.key(42), (_B, _V), jnp.float32)
    ga = g[:, :_K]
    m_tail = jnp.max(g[:, _K:], axis=1, keepdims=True)
    rank = (jnp.argmax(g[:, _K:], axis=1).astype(jnp.int32)
            + jnp.int32(_K)).reshape(_B, 1)

    temps = jnp.asarray(temperatures, jnp.float32).reshape(_B, 1)
    topk = jnp.asarray(top_ks, jnp.int32).reshape(_B, 1)
    topp = jnp.asarray(top_ps, jnp.float32).reshape(_B, 1)
    minp = jnp.asarray(min_ps, jnp.float32).reshape(_B, 1)
    need = jnp.asarray(need_min_p_sampling, jnp.int32).reshape(1, 1)

    grid = (_B // _RB,)
    col = pl.BlockSpec((_RB, 1), lambda i: (i, 0))
    out = pl.pallas_call(
        _sampler_block,
        grid=grid,
        in_specs=[
            pl.BlockSpec((_RB, _V), lambda i: (i, 0)),
            col, col, col, col,
            pl.BlockSpec((1, 1), lambda i: (0, 0)),
            pl.BlockSpec((_RB, _K), lambda i: (i, 0)),
            col, col,
        ],
        out_specs=col,
        out_shape=jax.ShapeDtypeStruct((_B, 1), jnp.int32),
        scratch_shapes=[pltpu.VMEM((_RB, _V), jnp.float32),
                        pltpu.VMEM((_RB, _V), jnp.int32)],
    )(logits[:, ::-1], temps, topk, topp, minp, need, ga, m_tail, rank)
    return out.reshape(-1)


# speedup vs baseline: 1.2209x; 1.2209x over previous
import jax
import jax.numpy as jnp
import numpy as np
from jax.experimental import pallas as pl
from jax.experimental.pallas import tpu as pltpu

_B = 64
_V = 100000
_K = 50
_RB = 8

_I32_MIN = np.int32(-2147483648)
_I32_MAX = np.int32(2147483647)
_NEG_INF = np.float32(-np.inf)


def _sortable(bits):
    return jnp.where(bits < 0, bits ^ jnp.int32(0x7FFFFFFF), bits)


def _unsortable(s):
    return jnp.where(s < 0, s ^ jnp.int32(0x7FFFFFFF), s)


def _mid(lo, hi):
    return (lo >> 1) + (hi >> 1) + (lo & hi & 1)


def _sampler_block(logits_ref, temp_ref, topk_ref, topp_ref, minp_ref,
                   need_ref, ga_ref, m_ref, rank_ref, out_ref, s_ref, e_ref):
    s_ref[...] = logits_ref[...] / temp_ref[...]
    idx = jax.lax.broadcasted_iota(jnp.int32, (1, _V), 1)

    topk = topk_ref[...]
    topp = topp_ref[...]
    minp = minp_ref[...]
    need = need_ref[...] != 0
    ga = ga_ref[...]
    m_tail = m_ref[...]
    rank = rank_ref[...]

    lane_k = jax.lax.broadcasted_iota(jnp.int32, (1, _K), 1)

    def val_body(_, carry):
        lo, hi, cnt_hi = carry
        mid = _mid(lo, hi)
        mid_f = jax.lax.bitcast_convert_type(_unsortable(mid), jnp.float32)
        cnt = jnp.sum((s_ref[...] > mid_f).astype(jnp.int32),
                      axis=1, keepdims=True)
        le = cnt <= rank
        return (jnp.where(le, lo, mid + 1), jnp.where(le, mid, hi),
                jnp.where(le, cnt, cnt_hi))

    lo0 = jnp.full((_RB, 1), np.int32(-2139095041))
    hi0 = jnp.full((_RB, 1), np.int32(2139095040))
    cnt0 = jnp.zeros((_RB, 1), jnp.int32)
    xstar, _, cnt_gt = jax.lax.fori_loop(0, 32, val_body, (lo0, hi0, cnt0))
    xstar_f = jax.lax.bitcast_convert_type(_unsortable(xstar), jnp.float32)
    k2 = rank - cnt_gt

    e_ref[...] = jnp.where(s_ref[...] == xstar_f, idx, jnp.int32(-1))

    def idx_body(_, carry):
        lo, hi = carry
        mid = _mid(lo, hi)
        cnt = jnp.sum((e_ref[...] > mid).astype(jnp.int32),
                      axis=1, keepdims=True)
        le = cnt <= k2
        return jnp.where(le, lo, mid + 1), jnp.where(le, mid, hi)

    lo0i = jnp.zeros((_RB, 1), jnp.int32)
    hi0i = jnp.full((_RB, 1), jnp.int32(131071))
    tstar, _ = jax.lax.fori_loop(0, 17, idx_body, (lo0i, hi0i))

    def extract_body(k, carry):
        prev_i, csum, v0, best_s, best_i = carry
        sv = jnp.where(idx == prev_i, _NEG_INF, s_ref[...])
        s_ref[...] = sv
        m = jnp.max(sv, axis=1, keepdims=True)
        im = jnp.max(jnp.where(sv == m, idx, jnp.int32(-1)),
                     axis=1, keepdims=True)
        csum = csum + m
        v0 = jnp.where(k == 0, m, v0)
        thr = jnp.where(need, v0 * minp, _NEG_INF)
        keep = (k < topk) & (csum - m <= topp) & (m >= thr)
        mval = jnp.where(keep, m, jnp.float32(0.0))
        gk = jnp.sum(jnp.where(lane_k == k, ga, jnp.float32(0.0)),
                     axis=1, keepdims=True)
        score = mval + gk
        upd = score > best_s
        best_s = jnp.where(upd, score, best_s)
        best_i = jnp.where(upd, im, best_i)
        return im, csum, v0, best_s, best_i

    zero_col = jnp.zeros((_RB, 1), jnp.float32)
    init = (jnp.full((_RB, 1), jnp.int32(-1)),
            zero_col, zero_col,
            jnp.full((_RB, 1), _NEG_INF),
            jnp.zeros((_RB, 1), jnp.int32))
    _, _, _, best_s, best_i = jax.lax.fori_loop(0, _K, extract_body, init)

    out_ref[...] = jnp.where(best_s >= m_tail, best_i, tstar)


def kernel(logits, temperatures, top_ks, top_ps, min_ps, need_min_p_sampling):
    logits = jnp.reshape(logits, (-1, logits.shape[-1]))

    g = jax.random.gumbel(jax.random.key(42), (_B, _V), jnp.float32)
    ga = g[:, :_K]
    m_tail = jnp.max(g[:, _K:], axis=1, keepdims=True)
    rank = (jnp.argmax(g[:, _K:], axis=1).astype(jnp.int32)
            + jnp.int32(_K)).reshape(_B, 1)

    temps = jnp.asarray(temperatures, jnp.float32).reshape(_B, 1)
    topk = jnp.asarray(top_ks, jnp.int32).reshape(_B, 1)
    topp = jnp.asarray(top_ps, jnp.float32).reshape(_B, 1)
    minp = jnp.asarray(min_ps, jnp.float32).reshape(_B, 1)
    need = jnp.asarray(need_min_p_sampling, jnp.int32).reshape(1, 1)

    grid = (_B // _RB,)
    col = pl.BlockSpec((_RB, 1), lambda i: (i, 0))
    out = pl.pallas_call(
        _sampler_block,
        grid=grid,
        in_specs=[
            pl.BlockSpec((_RB, _V), lambda i: (i, 0)),
            col, col, col, col,
            pl.BlockSpec((1, 1), lambda i: (0, 0)),
            pl.BlockSpec((_RB, _K), lambda i: (i, 0)),
            col, col,
        ],
        out_specs=col,
        out_shape=jax.ShapeDtypeStruct((_B, 1), jnp.int32),
        scratch_shapes=[pltpu.VMEM((_RB, _V), jnp.float32),
                        pltpu.VMEM((_RB, _V), jnp.int32)],
    )(logits, temps, topk, topp, minp, need, ga, m_tail, rank)
    return out.reshape(-1)


# speedup vs baseline: 1.6626x; 1.3618x over previous
import jax
import jax.numpy as jnp
import numpy as np
from jax.experimental import pallas as pl
from jax.experimental.pallas import tpu as pltpu

_B = 64
_V = 100000
_K = 50
_RB = 16

_I32_MIN = np.int32(-2147483648)
_I32_MAX = np.int32(2147483647)
_NEG_INF = np.float32(-np.inf)


def _sortable(bits):
    return jnp.where(bits < 0, bits ^ jnp.int32(0x7FFFFFFF), bits)


def _unsortable(s):
    return jnp.where(s < 0, s ^ jnp.int32(0x7FFFFFFF), s)


def _mid(lo, hi):
    return (lo >> 1) + (hi >> 1) + (lo & hi & 1)


def _sampler_block(logits_ref, temp_ref, topk_ref, topp_ref, minp_ref,
                   need_ref, ga_ref, m_ref, rank_ref, out_ref, s_ref):
    x = logits_ref[...] / temp_ref[...]
    s_ref[...] = _sortable(jax.lax.bitcast_convert_type(x, jnp.int32))
    idx = jax.lax.broadcasted_iota(jnp.int32, (1, _V), 1)

    topk = topk_ref[...]
    topp = topp_ref[...]
    minp = minp_ref[...]
    need = need_ref[...] != 0
    ga = ga_ref[...]
    m_tail = m_ref[...]
    rank = rank_ref[...]

    lane_k = jax.lax.broadcasted_iota(jnp.int32, (1, _K), 1)

    def val_body(_, carry):
        lo, hi, cnt_hi = carry
        mid = _mid(lo, hi)
        cnt = jnp.sum((s_ref[...] > mid).astype(jnp.int32),
                      axis=1, keepdims=True)
        le = cnt <= rank
        return (jnp.where(le, lo, mid + 1), jnp.where(le, mid, hi),
                jnp.where(le, cnt, cnt_hi))

    lo0 = jnp.full((_RB, 1), _I32_MIN)
    hi0 = jnp.full((_RB, 1), _I32_MAX)
    cnt0 = jnp.zeros((_RB, 1), jnp.int32)
    xstar, _, cnt_gt = jax.lax.fori_loop(0, 32, val_body, (lo0, hi0, cnt0))
    k2 = rank - cnt_gt

    def idx_body(_, carry):
        lo, hi = carry
        mid = _mid(lo, hi)
        cnt = jnp.sum(((s_ref[...] == xstar) & (idx > mid)).astype(jnp.int32),
                      axis=1, keepdims=True)
        le = cnt <= k2
        return jnp.where(le, lo, mid + 1), jnp.where(le, mid, hi)

    lo0i = jnp.zeros((_RB, 1), jnp.int32)
    hi0i = jnp.full((_RB, 1), jnp.int32(131071))
    tstar, _ = jax.lax.fori_loop(0, 17, idx_body, (lo0i, hi0i))

    def extract_body(k, carry):
        prev_i, csum, v0, best_s, best_i = carry
        sv = jnp.where(idx == prev_i, _I32_MIN, s_ref[...])
        s_ref[...] = sv
        sm = jnp.max(sv, axis=1, keepdims=True)
        im = jnp.max(jnp.where(sv == sm, idx, jnp.int32(-1)),
                     axis=1, keepdims=True)
        m = jax.lax.bitcast_convert_type(_unsortable(sm), jnp.float32)
        csum = csum + m
        v0 = jnp.where(k == 0, m, v0)
        thr = jnp.where(need, v0 * minp, _NEG_INF)
        keep = (k < topk) & (csum - m <= topp) & (m >= thr)
        mval = jnp.where(keep, m, jnp.float32(0.0))
        gk = jnp.sum(jnp.where(lane_k == k, ga, jnp.float32(0.0)),
                     axis=1, keepdims=True)
        score = mval + gk
        upd = score > best_s
        best_s = jnp.where(upd, score, best_s)
        best_i = jnp.where(upd, im, best_i)
        return im, csum, v0, best_s, best_i

    zero_col = jnp.zeros((_RB, 1), jnp.float32)
    init = (jnp.full((_RB, 1), jnp.int32(-1)),
            zero_col, zero_col,
            jnp.full((_RB, 1), _NEG_INF),
            jnp.zeros((_RB, 1), jnp.int32))
    _, _, _, best_s, best_i = jax.lax.fori_loop(0, _K, extract_body, init)

    out_ref[...] = jnp.where(best_s >= m_tail, best_i, tstar)


def kernel(logits, temperatures, top_ks, top_ps, min_ps, need_min_p_sampling):
    logits = jnp.reshape(logits, (-1, logits.shape[-1]))

    g = jax.random.gumbel(jax.random.key(42), (_B, _V), jnp.float32)
    ga = g[:, :_K]
    m_tail = jnp.max(g[:, _K:], axis=1, keepdims=True)
    rank = (jnp.argmax(g[:, _K:], axis=1).astype(jnp.int32)
            + jnp.int32(_K)).reshape(_B, 1)

    temps = jnp.asarray(temperatures, jnp.float32).reshape(_B, 1)
    topk = jnp.asarray(top_ks, jnp.int32).reshape(_B, 1)
    topp = jnp.asarray(top_ps, jnp.float32).reshape(_B, 1)
    minp = jnp.asarray(min_ps, jnp.float32).reshape(_B, 1)
    need = jnp.asarray(need_min_p_sampling, jnp.int32).reshape(1, 1)

    grid = (_B // _RB,)
    col = pl.BlockSpec((_RB, 1), lambda i: (i, 0))
    out = pl.pallas_call(
        _sampler_block,
        grid=grid,
        in_specs=[
            pl.BlockSpec((_RB, _V), lambda i: (i, 0)),
            col, col, col, col,
            pl.BlockSpec((1, 1), lambda i: (0, 0)),
            pl.BlockSpec((_RB, _K), lambda i: (i, 0)),
            col, col,
        ],
        out_specs=col,
        out_shape=jax.ShapeDtypeStruct((_B, 1), jnp.int32),
        scratch_shapes=[pltpu.VMEM((_RB, _V), jnp.int32)],
    )(logits, temps, topk, topp, minp, need, ga, m_tail, rank)
    return out.reshape(-1)


# speedup vs baseline: 1.8007x; 1.0830x over previous
import jax
import jax.numpy as jnp
import numpy as np
from jax.experimental import pallas as pl
from jax.experimental.pallas import tpu as pltpu

_B = 64
_V = 100000
_K = 50
_RB = 32

_I32_MIN = np.int32(-2147483648)
_I32_MAX = np.int32(2147483647)
_NEG_INF = np.float32(-np.inf)


def _sortable(bits):
    return jnp.where(bits < 0, bits ^ jnp.int32(0x7FFFFFFF), bits)


def _unsortable(s):
    return jnp.where(s < 0, s ^ jnp.int32(0x7FFFFFFF), s)


def _mid(lo, hi):
    return (lo >> 1) + (hi >> 1) + (lo & hi & 1)


def _sampler_block(logits_ref, temp_ref, topk_ref, topp_ref, minp_ref,
                   need_ref, ga_ref, m_ref, rank_ref, out_ref, s_ref):
    x = logits_ref[...] / temp_ref[...]
    s_ref[...] = _sortable(jax.lax.bitcast_convert_type(x, jnp.int32))
    idx = jax.lax.broadcasted_iota(jnp.int32, (1, _V), 1)

    topk = topk_ref[...]
    topp = topp_ref[...]
    minp = minp_ref[...]
    need = need_ref[...] != 0
    ga = ga_ref[...]
    m_tail = m_ref[...]
    rank = rank_ref[...]

    lane_k = jax.lax.broadcasted_iota(jnp.int32, (1, _K), 1)

    def val_body(_, carry):
        lo, hi, cnt_hi = carry
        mid = _mid(lo, hi)
        cnt = jnp.sum((s_ref[...] > mid).astype(jnp.int32),
                      axis=1, keepdims=True)
        le = cnt <= rank
        return (jnp.where(le, lo, mid + 1), jnp.where(le, mid, hi),
                jnp.where(le, cnt, cnt_hi))

    lo0 = jnp.full((_RB, 1), _I32_MIN)
    hi0 = jnp.full((_RB, 1), _I32_MAX)
    cnt0 = jnp.zeros((_RB, 1), jnp.int32)
    xstar, _, cnt_gt = jax.lax.fori_loop(0, 32, val_body, (lo0, hi0, cnt0))
    k2 = rank - cnt_gt

    def idx_body(_, carry):
        lo, hi = carry
        mid = _mid(lo, hi)
        cnt = jnp.sum(((s_ref[...] == xstar) & (idx > mid)).astype(jnp.int32),
                      axis=1, keepdims=True)
        le = cnt <= k2
        return jnp.where(le, lo, mid + 1), jnp.where(le, mid, hi)

    lo0i = jnp.zeros((_RB, 1), jnp.int32)
    hi0i = jnp.full((_RB, 1), jnp.int32(131071))
    tstar, _ = jax.lax.fori_loop(0, 17, idx_body, (lo0i, hi0i))

    def extract_body(k, carry):
        prev_i, csum, v0, best_s, best_i = carry
        sv = jnp.where(idx == prev_i, _I32_MIN, s_ref[...])
        s_ref[...] = sv
        sm = jnp.max(sv, axis=1, keepdims=True)
        im = jnp.max(jnp.where(sv == sm, idx, jnp.int32(-1)),
                     axis=1, keepdims=True)
        m = jax.lax.bitcast_convert_type(_unsortable(sm), jnp.float32)
        csum = csum + m
        v0 = jnp.where(k == 0, m, v0)
        thr = jnp.where(need, v0 * minp, _NEG_INF)
        keep = (k < topk) & (csum - m <= topp) & (m >= thr)
        mval = jnp.where(keep, m, jnp.float32(0.0))
        gk = jnp.sum(jnp.where(lane_k == k, ga, jnp.float32(0.0)),
                     axis=1, keepdims=True)
        score = mval + gk
        upd = score > best_s
        best_s = jnp.where(upd, score, best_s)
        best_i = jnp.where(upd, im, best_i)
        return im, csum, v0, best_s, best_i

    zero_col = jnp.zeros((_RB, 1), jnp.float32)
    init = (jnp.full((_RB, 1), jnp.int32(-1)),
            zero_col, zero_col,
            jnp.full((_RB, 1), _NEG_INF),
            jnp.zeros((_RB, 1), jnp.int32))
    _, _, _, best_s, best_i = jax.lax.fori_loop(0, _K, extract_body, init)

    out_ref[...] = jnp.where(best_s >= m_tail, best_i, tstar)


def kernel(logits, temperatures, top_ks, top_ps, min_ps, need_min_p_sampling):
    logits = jnp.reshape(logits, (-1, logits.shape[-1]))

    g = jax.random.gumbel(jax.random.key(42), (_B, _V), jnp.float32)
    ga = g[:, :_K]
    m_tail = jnp.max(g[:, _K:], axis=1, keepdims=True)
    rank = (jnp.argmax(g[:, _K:], axis=1).astype(jnp.int32)
            + jnp.int32(_K)).reshape(_B, 1)

    temps = jnp.asarray(temperatures, jnp.float32).reshape(_B, 1)
    topk = jnp.asarray(top_ks, jnp.int32).reshape(_B, 1)
    topp = jnp.asarray(top_ps, jnp.float32).reshape(_B, 1)
    minp = jnp.asarray(min_ps, jnp.float32).reshape(_B, 1)
    need = jnp.asarray(need_min_p_sampling, jnp.int32).reshape(1, 1)

    grid = (_B // _RB,)
    col = pl.BlockSpec((_RB, 1), lambda i: (i, 0))
    out = pl.pallas_call(
        _sampler_block,
        grid=grid,
        in_specs=[
            pl.BlockSpec((_RB, _V), lambda i: (i, 0)),
            col, col, col, col,
            pl.BlockSpec((1, 1), lambda i: (0, 0)),
            pl.BlockSpec((_RB, _K), lambda i: (i, 0)),
            col, col,
        ],
        out_specs=col,
        out_shape=jax.ShapeDtypeStruct((_B, 1), jnp.int32),
        scratch_shapes=[pltpu.VMEM((_RB, _V), jnp.int32)],
    )(logits, temps, topk, topp, minp, need, ga, m_tail, rank)
    return out.reshape(-1)
